# baseline (speedup 1.0000x reference)
import functools

import jax
import jax.numpy as jnp
from jax import lax
from jax.experimental import pallas as pl
from jax.experimental.pallas import tpu as pltpu
from jax.experimental.pallas import tpu_sc as plsc

NC = 2
NS = 16
NW = NC * NS
K = 120
CM = 12
DW = 8


def _mesh():
    return plsc.VectorSubcoreMesh(
        core_axis_name="c", subcore_axis_name="s", num_cores=NC, num_subcores=NS
    )


_SC_PARAMS = pltpu.CompilerParams(use_tc_tiling_on_sc=False)


def _deg_call(Np, C):
    rows_t = Np // NS

    @functools.partial(
        pl.kernel,
        out_type=jax.ShapeDtypeStruct((NC, Np, DW), jnp.float32),
        mesh=_mesh(),
        compiler_params=_SC_PARAMS,
        scratch_types=[
            pltpu.VMEM((C, K), jnp.int32),
            pltpu.VMEM((K, DW), jnp.float32),
            pltpu.VMEM_SHARED((Np, DW), jnp.float32),
            pltpu.SemaphoreType.DMA,
        ],
    )
    def deg_k(dst_hbm, zeros_hbm, ones_hbm, out_hbm, dst_v, ones_v, acc, sem):
        c = lax.axis_index("c")
        s = lax.axis_index("s")
        t = c * NS + s
        pltpu.sync_copy(
            zeros_hbm.at[pl.ds(s * rows_t, rows_t)], acc.at[pl.ds(s * rows_t, rows_t)]
        )
        pltpu.sync_copy(dst_hbm.at[pl.ds(t * C, C)], dst_v)
        pltpu.sync_copy(ones_hbm, ones_v)
        plsc.subcore_barrier()

        def body(j, carry):
            pltpu.async_copy(ones_v, acc.at[dst_v.at[j]], sem, add=True)
            return carry

        lax.fori_loop(0, C, body, 0)

        def drain(j, carry):
            pltpu.make_async_copy(ones_v, acc.at[dst_v.at[0]], sem).wait()
            return carry

        lax.fori_loop(0, C, drain, 0)
        plsc.subcore_barrier()
        pltpu.sync_copy(
            acc.at[pl.ds(s * rows_t, rows_t)],
            out_hbm.at[c, pl.ds(s * rows_t, rows_t)],
        )

    return deg_k


def _agg_call(Np, C, Wd, nb, d, g):
    rows_t = Np // NS
    assert C % g == 0 and g % nb == 0 and 0 < d < nb <= g
    ng = C // g

    @functools.partial(
        pl.kernel,
        out_type=jax.ShapeDtypeStruct((NC, Np, Wd), jnp.float32),
        mesh=_mesh(),
        compiler_params=_SC_PARAMS,
        scratch_types=[
            pltpu.VMEM((g, K), jnp.int32),
            pltpu.VMEM((g, K), jnp.int32),
            [pltpu.VMEM((K, Wd), jnp.float32) for _ in range(nb)],
            pltpu.VMEM_SHARED((Np, Wd), jnp.float32),
            [pltpu.SemaphoreType.DMA for _ in range(nb)],
            [pltpu.SemaphoreType.DMA for _ in range(nb)],
        ],
    )
    def agg_k(y_hbm, src_hbm, dst_hbm, out_hbm, src_v, dst_v, bufs, acc, gsem, ssem):
        c = lax.axis_index("c")
        s = lax.axis_index("s")
        t = c * NS + s
        pltpu.sync_copy(
            y_hbm.at[pl.ds(s * rows_t, rows_t)], acc.at[pl.ds(s * rows_t, rows_t)]
        )
        plsc.subcore_barrier()

        def group(gi, carry):
            base = t * C + gi * g
            pltpu.sync_copy(src_hbm.at[pl.ds(base, g)], src_v)
            pltpu.sync_copy(dst_hbm.at[pl.ds(base, g)], dst_v)
            for j0 in range(d):
                pltpu.async_copy(y_hbm.at[src_v.at[j0]], bufs[j0 % nb], gsem[j0 % nb])
            for j in range(g):
                b = j % nb
                nxt = j + d
                if nxt < g:
                    bn = nxt % nb
                    if nxt >= nb:
                        pltpu.make_async_copy(
                            bufs[bn], acc.at[dst_v.at[0]], ssem[bn]
                        ).wait()
                    pltpu.async_copy(y_hbm.at[src_v.at[nxt]], bufs[bn], gsem[bn])
                pltpu.make_async_copy(y_hbm.at[src_v.at[0]], bufs[b], gsem[b]).wait()
                pltpu.async_copy(bufs[b], acc.at[dst_v.at[j]], ssem[b], add=True)
            for b in range(nb):
                pltpu.make_async_copy(bufs[b], acc.at[dst_v.at[0]], ssem[b]).wait()
            return carry

        lax.fori_loop(0, ng, group, 0)
        plsc.subcore_barrier()
        pltpu.sync_copy(
            acc.at[pl.ds(s * rows_t, rows_t)],
            out_hbm.at[c, pl.ds(s * rows_t, rows_t)],
        )

    return agg_k


def _dinv_of(degp_ref):
    d = degp_ref[0, :, 0:1] + degp_ref[1, :, 0:1] + 1.0
    return lax.rsqrt(d)


def _xw_call(xp, W1):
    Np, D = xp.shape
    H = W1.shape[1]
    BN = Np // 8

    def body(x_ref, w_ref, y_ref):
        y_ref[...] = jnp.dot(
            x_ref[...], w_ref[...], preferred_element_type=jnp.float32
        )

    return pl.pallas_call(
        body,
        grid=(8,),
        in_specs=[
            pl.BlockSpec((BN, D), lambda i: (i, 0)),
            pl.BlockSpec((D, H), lambda i: (0, 0)),
        ],
        out_specs=pl.BlockSpec((BN, H), lambda i: (i, 0)),
        out_shape=jax.ShapeDtypeStruct((Np, H), jnp.float32),
    )(xp, W1)


def _scale_call(degp, xw):
    _, Np, _ = degp.shape
    H = xw.shape[1]
    BN = Np // 8

    def body(degp_ref, xw_ref, y_ref):
        y_ref[...] = _dinv_of(degp_ref) * xw_ref[...]

    return pl.pallas_call(
        body,
        grid=(8,),
        in_specs=[
            pl.BlockSpec((NC, BN, DW), lambda i: (0, i, 0)),
            pl.BlockSpec((BN, H), lambda i: (i, 0)),
        ],
        out_specs=pl.BlockSpec((BN, H), lambda i: (i, 0)),
        out_shape=jax.ShapeDtypeStruct((Np, H), jnp.float32),
    )(degp, xw)


def _mid_call(p, y1, degp, W2p, b1r):
    _, Np, H = p.shape
    BN = Np // 8

    def body(p_ref, y1_ref, degp_ref, w_ref, b_ref, out_ref):
        dinv = _dinv_of(degp_ref)
        agg = p_ref[0] + p_ref[1] - y1_ref[...]
        h = jnp.maximum(dinv * agg + b_ref[...], 0.0)
        hw = jnp.dot(h, w_ref[...], preferred_element_type=jnp.float32)
        out_ref[...] = dinv * hw

    return pl.pallas_call(
        body,
        grid=(8,),
        in_specs=[
            pl.BlockSpec((NC, BN, H), lambda i: (0, i, 0)),
            pl.BlockSpec((BN, H), lambda i: (i, 0)),
            pl.BlockSpec((NC, BN, DW), lambda i: (0, i, 0)),
            pl.BlockSpec((H, DW), lambda i: (0, 0)),
            pl.BlockSpec((1, H), lambda i: (0, 0)),
        ],
        out_specs=pl.BlockSpec((BN, DW), lambda i: (i, 0)),
        out_shape=jax.ShapeDtypeStruct((Np, DW), jnp.float32),
    )(p, y1, degp, W2p, b1r)


def _final_call(q, y2, degp, b2r, nout, n):
    _, Np, _ = q.shape
    BN = Np // 8

    def body(q_ref, y2_ref, degp_ref, b_ref, out_ref):
        dinv = _dinv_of(degp_ref)
        o = dinv * (q_ref[0] + q_ref[1] - y2_ref[...]) + b_ref[...]
        lane = lax.broadcasted_iota(jnp.int32, o.shape, 1)
        valid = lane < nout
        neg = jnp.full_like(o, -jnp.inf)
        om = jnp.where(valid, o, neg)
        m = jnp.max(om, axis=1, keepdims=True)
        e = jnp.where(valid, jnp.exp(om - m), 0.0)
        ssum = jnp.sum(e, axis=1, keepdims=True)
        out_ref[...] = (e / ssum)[:, :nout]

    return pl.pallas_call(
        body,
        grid=(8,),
        in_specs=[
            pl.BlockSpec((NC, BN, DW), lambda i: (0, i, 0)),
            pl.BlockSpec((BN, DW), lambda i: (i, 0)),
            pl.BlockSpec((NC, BN, DW), lambda i: (0, i, 0)),
            pl.BlockSpec((1, DW), lambda i: (0, 0)),
        ],
        out_specs=pl.BlockSpec((BN, nout), lambda i: (i, 0)),
        out_shape=jax.ShapeDtypeStruct((n, nout), jnp.float32),
    )(q, y2, degp, b2r)


def kernel(x, edge_index, W1, b1, W2, b2):
    N, D = x.shape
    H = W1.shape[1]
    OUT = W2.shape[1]
    E = edge_index.shape[1]
    assert D % 128 == 0 and H % 128 == 0 and OUT <= DW

    Np = -(-N // 128) * 128
    if Np == N:
        Np += 128

    src = edge_index[0].astype(jnp.int32)
    dst = edge_index[1].astype(jnp.int32)
    EC = NW * K * CM
    Et = -(-E // EC) * EC
    if Et != E:
        pad = N + (jnp.arange(Et - E, dtype=jnp.int32) % (Np - N))
        src = jnp.concatenate([src, pad])
        dst = jnp.concatenate([dst, pad])
    C = Et // (NW * K)
    srcr = src.reshape(Et // K, K)
    dstr = dst.reshape(Et // K, K)

    xp = jnp.pad(x, ((0, Np - N), (0, 0)))
    W2p = jnp.pad(W2, ((0, 0), (0, DW - OUT)))
    b1r = b1.reshape(1, H)
    b2r = jnp.pad(b2, (0, DW - OUT)).reshape(1, DW)
    zeros_dw = jnp.zeros((Np, DW), jnp.float32)
    ones_dw = jnp.ones((K, DW), jnp.float32)

    xw = _xw_call(xp, W1)
    degp = _deg_call(Np, C)(dstr, zeros_dw, ones_dw)
    y1 = _scale_call(degp, xw)
    p = _agg_call(Np, C, H, 3, 2, 12)(y1, srcr, dstr)
    y2 = _mid_call(p, y1, degp, W2p, b1r)
    q = _agg_call(Np, C, DW, 6, 3, 12)(y2, srcr, dstr)
    return _final_call(q, y2, degp, b2r, OUT, N)


# speedup vs baseline: 1.0044x; 1.0044x over previous
import functools

import jax
import jax.numpy as jnp
from jax import lax
from jax.experimental import pallas as pl
from jax.experimental.pallas import tpu as pltpu
from jax.experimental.pallas import tpu_sc as plsc

NC = 2
NS = 16
NW = NC * NS
K = 120
CM = 12
DW = 8


def _mesh():
    return plsc.VectorSubcoreMesh(
        core_axis_name="c", subcore_axis_name="s", num_cores=NC, num_subcores=NS
    )


_SC_PARAMS = pltpu.CompilerParams(use_tc_tiling_on_sc=False)


def _deg_call(Np, C):
    rows_t = Np // NS

    @functools.partial(
        pl.kernel,
        out_type=jax.ShapeDtypeStruct((NC, Np, DW), jnp.float32),
        mesh=_mesh(),
        compiler_params=_SC_PARAMS,
        scratch_types=[
            pltpu.VMEM((C, K), jnp.int32),
            pltpu.VMEM((K, DW), jnp.float32),
            pltpu.VMEM_SHARED((Np, DW), jnp.float32),
            pltpu.SemaphoreType.DMA,
        ],
    )
    def deg_k(dst_hbm, zeros_hbm, ones_hbm, out_hbm, dst_v, ones_v, acc, sem):
        c = lax.axis_index("c")
        s = lax.axis_index("s")
        t = c * NS + s
        pltpu.sync_copy(
            zeros_hbm.at[pl.ds(s * rows_t, rows_t)], acc.at[pl.ds(s * rows_t, rows_t)]
        )
        pltpu.sync_copy(dst_hbm.at[pl.ds(t * C, C)], dst_v)
        pltpu.sync_copy(ones_hbm, ones_v)
        plsc.subcore_barrier()

        def body(j, carry):
            pltpu.async_copy(ones_v, acc.at[dst_v.at[j]], sem, add=True)
            return carry

        lax.fori_loop(0, C, body, 0)

        def drain(j, carry):
            pltpu.make_async_copy(ones_v, acc.at[dst_v.at[0]], sem).wait()
            return carry

        lax.fori_loop(0, C, drain, 0)
        plsc.subcore_barrier()
        pltpu.sync_copy(
            acc.at[pl.ds(s * rows_t, rows_t)],
            out_hbm.at[c, pl.ds(s * rows_t, rows_t)],
        )

    return deg_k


def _agg_call(Np, C, Wd, nb, d, g):
    rows_t = Np // NS
    assert C % g == 0 and g % nb == 0 and 0 < d < nb <= g
    ng = C // g

    @functools.partial(
        pl.kernel,
        out_type=jax.ShapeDtypeStruct((NC, Np, Wd), jnp.float32),
        mesh=_mesh(),
        compiler_params=_SC_PARAMS,
        scratch_types=[
            pltpu.VMEM((g, K), jnp.int32),
            pltpu.VMEM((g, K), jnp.int32),
            [pltpu.VMEM((K, Wd), jnp.float32) for _ in range(nb)],
            pltpu.VMEM_SHARED((Np, Wd), jnp.float32),
            [pltpu.SemaphoreType.DMA for _ in range(nb)],
            [pltpu.SemaphoreType.DMA for _ in range(nb)],
        ],
    )
    def agg_k(y_hbm, src_hbm, dst_hbm, out_hbm, src_v, dst_v, bufs, acc, gsem, ssem):
        c = lax.axis_index("c")
        s = lax.axis_index("s")
        t = c * NS + s
        pltpu.sync_copy(
            y_hbm.at[pl.ds(s * rows_t, rows_t)], acc.at[pl.ds(s * rows_t, rows_t)]
        )
        plsc.subcore_barrier()

        def group(gi, carry):
            base = t * C + gi * g
            pltpu.sync_copy(src_hbm.at[pl.ds(base, g)], src_v)
            pltpu.sync_copy(dst_hbm.at[pl.ds(base, g)], dst_v)
            for j0 in range(d):
                pltpu.async_copy(y_hbm.at[src_v.at[j0]], bufs[j0 % nb], gsem[j0 % nb])
            for j in range(g):
                b = j % nb
                nxt = j + d
                if nxt < g:
                    bn = nxt % nb
                    if nxt >= nb:
                        pltpu.make_async_copy(
                            bufs[bn], acc.at[dst_v.at[0]], ssem[bn]
                        ).wait()
                    pltpu.async_copy(y_hbm.at[src_v.at[nxt]], bufs[bn], gsem[bn])
                pltpu.make_async_copy(y_hbm.at[src_v.at[0]], bufs[b], gsem[b]).wait()
                pltpu.async_copy(bufs[b], acc.at[dst_v.at[j]], ssem[b], add=True)
            for b in range(nb):
                pltpu.make_async_copy(bufs[b], acc.at[dst_v.at[0]], ssem[b]).wait()
            return carry

        lax.fori_loop(0, ng, group, 0)
        plsc.subcore_barrier()
        pltpu.sync_copy(
            acc.at[pl.ds(s * rows_t, rows_t)],
            out_hbm.at[c, pl.ds(s * rows_t, rows_t)],
        )

    return agg_k


def _dinv_of(degp_ref):
    d = degp_ref[0, :, 0:1] + degp_ref[1, :, 0:1] + 1.0
    return lax.rsqrt(d)


def _xw_call(xp, W1):
    Np, D = xp.shape
    H = W1.shape[1]
    BN = Np // 8

    def body(x_ref, w_ref, y_ref):
        y_ref[...] = jnp.dot(
            x_ref[...], w_ref[...], preferred_element_type=jnp.float32
        )

    return pl.pallas_call(
        body,
        grid=(8,),
        in_specs=[
            pl.BlockSpec((BN, D), lambda i: (i, 0)),
            pl.BlockSpec((D, H), lambda i: (0, 0)),
        ],
        out_specs=pl.BlockSpec((BN, H), lambda i: (i, 0)),
        out_shape=jax.ShapeDtypeStruct((Np, H), jnp.float32),
    )(xp, W1)


def _scale_call(degp, xw):
    _, Np, _ = degp.shape
    H = xw.shape[1]
    BN = Np // 8

    def body(degp_ref, xw_ref, y_ref):
        y_ref[...] = _dinv_of(degp_ref) * xw_ref[...]

    return pl.pallas_call(
        body,
        grid=(8,),
        in_specs=[
            pl.BlockSpec((NC, BN, DW), lambda i: (0, i, 0)),
            pl.BlockSpec((BN, H), lambda i: (i, 0)),
        ],
        out_specs=pl.BlockSpec((BN, H), lambda i: (i, 0)),
        out_shape=jax.ShapeDtypeStruct((Np, H), jnp.float32),
    )(degp, xw)


def _mid_call(p, y1, degp, W2p, b1r):
    _, Np, H = p.shape
    BN = Np // 8

    def body(p_ref, y1_ref, degp_ref, w_ref, b_ref, out_ref):
        dinv = _dinv_of(degp_ref)
        agg = p_ref[0] + p_ref[1] - y1_ref[...]
        h = jnp.maximum(dinv * agg + b_ref[...], 0.0)
        hw = jnp.dot(h, w_ref[...], preferred_element_type=jnp.float32)
        out_ref[...] = dinv * hw

    return pl.pallas_call(
        body,
        grid=(8,),
        in_specs=[
            pl.BlockSpec((NC, BN, H), lambda i: (0, i, 0)),
            pl.BlockSpec((BN, H), lambda i: (i, 0)),
            pl.BlockSpec((NC, BN, DW), lambda i: (0, i, 0)),
            pl.BlockSpec((H, DW), lambda i: (0, 0)),
            pl.BlockSpec((1, H), lambda i: (0, 0)),
        ],
        out_specs=pl.BlockSpec((BN, DW), lambda i: (i, 0)),
        out_shape=jax.ShapeDtypeStruct((Np, DW), jnp.float32),
    )(p, y1, degp, W2p, b1r)


def _final_call(q, y2, degp, b2r, nout, n):
    _, Np, _ = q.shape
    BN = Np // 8

    def body(q_ref, y2_ref, degp_ref, b_ref, out_ref):
        dinv = _dinv_of(degp_ref)
        o = dinv * (q_ref[0] + q_ref[1] - y2_ref[...]) + b_ref[...]
        lane = lax.broadcasted_iota(jnp.int32, o.shape, 1)
        valid = lane < nout
        neg = jnp.full_like(o, -jnp.inf)
        om = jnp.where(valid, o, neg)
        m = jnp.max(om, axis=1, keepdims=True)
        e = jnp.where(valid, jnp.exp(om - m), 0.0)
        ssum = jnp.sum(e, axis=1, keepdims=True)
        out_ref[...] = (e / ssum)[:, :nout]

    return pl.pallas_call(
        body,
        grid=(8,),
        in_specs=[
            pl.BlockSpec((NC, BN, DW), lambda i: (0, i, 0)),
            pl.BlockSpec((BN, DW), lambda i: (i, 0)),
            pl.BlockSpec((NC, BN, DW), lambda i: (0, i, 0)),
            pl.BlockSpec((1, DW), lambda i: (0, 0)),
        ],
        out_specs=pl.BlockSpec((BN, nout), lambda i: (i, 0)),
        out_shape=jax.ShapeDtypeStruct((n, nout), jnp.float32),
    )(q, y2, degp, b2r)


def kernel(x, edge_index, W1, b1, W2, b2):
    N, D = x.shape
    H = W1.shape[1]
    OUT = W2.shape[1]
    E = edge_index.shape[1]
    assert D % 128 == 0 and H % 128 == 0 and OUT <= DW

    Np = -(-N // 128) * 128
    if Np == N:
        Np += 128

    src = edge_index[0].astype(jnp.int32)
    dst = edge_index[1].astype(jnp.int32)
    EC = NW * K * CM
    Et = -(-E // EC) * EC
    if Et != E:
        pad = N + (jnp.arange(Et - E, dtype=jnp.int32) % (Np - N))
        src = jnp.concatenate([src, pad])
        dst = jnp.concatenate([dst, pad])
    C = Et // (NW * K)
    srcr = src.reshape(Et // K, K)
    dstr = dst.reshape(Et // K, K)

    xp = jnp.pad(x, ((0, Np - N), (0, 0)))
    W2p = jnp.pad(W2, ((0, 0), (0, DW - OUT)))
    b1r = b1.reshape(1, H)
    b2r = jnp.pad(b2, (0, DW - OUT)).reshape(1, DW)
    zeros_dw = jnp.zeros((Np, DW), jnp.float32)
    ones_dw = jnp.ones((K, DW), jnp.float32)

    xw = _xw_call(xp, W1)
    degp = _deg_call(Np, C)(dstr, zeros_dw, ones_dw)
    y1 = _scale_call(degp, xw)
    p = _agg_call(Np, C, H, 3, 2, 12)(y1, srcr, dstr)
    y2 = _mid_call(p, y1, degp, W2p, b1r)
    q = _agg_call(Np, C, DW, 6, 4, 12)(y2, srcr, dstr)
    return _final_call(q, y2, degp, b2r, OUT, N)
